# baseline (speedup 1.0000x reference)
import functools

import jax
import jax.numpy as jnp
from jax import lax
from jax.experimental import pallas as pl
from jax.experimental.pallas import tpu as pltpu
from jax.experimental.pallas import tpu_sc as plsc

_NUM_WORKERS = 32
_NBUF = 3


def kernel(inputs, token_table, pos_table):
    B, L = inputs.shape
    V, E = token_table.shape
    BBLK = B // _NUM_WORKERS

    inputs_t = jnp.swapaxes(inputs, 0, 1)

    mesh = plsc.VectorSubcoreMesh(core_axis_name="c", subcore_axis_name="s")

    @functools.partial(
        pl.kernel,
        mesh=mesh,
        compiler_params=pltpu.CompilerParams(use_tc_tiling_on_sc=False,
                                             needs_layout_passes=False),
        out_type=jax.ShapeDtypeStruct((L, E, B), jnp.float32),
        scratch_types=[
            pltpu.VMEM((L, BBLK), jnp.int32),
            pltpu.VMEM((L, E), jnp.float32),
            [pltpu.VMEM((BBLK, E), jnp.float32)] * _NBUF,
            [pltpu.VMEM((E, BBLK), jnp.float32)] * _NBUF,
            [pltpu.SemaphoreType.DMA] * _NBUF,
            [pltpu.SemaphoreType.DMA] * _NBUF,
        ],
    )
    def emb_kernel(inputs_hbm, table_hbm, pos_hbm, out_hbm,
                   idx_v, pos_v, gbufs, tbufs, gsems, wsems):
        wid = lax.axis_index("s") * 2 + lax.axis_index("c")
        bbase = wid * BBLK

        pltpu.sync_copy(inputs_hbm.at[:, pl.ds(bbase, BBLK)], idx_v)
        pltpu.sync_copy(pos_hbm, pos_v)

        def gather_cp(l, k):
            return pltpu.make_async_copy(
                table_hbm.at[idx_v.at[l, :]], gbufs[k], gsems[k])

        def wb_cp(l, k):
            return pltpu.make_async_copy(
                tbufs[k], out_hbm.at[l, :, pl.ds(bbase, BBLK)], wsems[k])

        for k in range(_NBUF - 1):
            gather_cp(k, k).start()

        lanes = jnp.arange(16, dtype=jnp.int32)
        lane_groups = [lanes + (16 * g) for g in range(BBLK // 16)]

        def body(l, carry):
            k = lax.rem(l, _NBUF)

            @pl.when(l + _NBUF - 1 < L)
            def _fire_ahead():
                kn = lax.rem(l + _NBUF - 1, _NBUF)
                for kk in range(_NBUF):
                    @pl.when(kn == kk)
                    def _fire():
                        gather_cp(l + _NBUF - 1, kk).start()

            for kk in range(_NBUF):
                @pl.when(k == kk)
                def _work():
                    gather_cp(l, kk).wait()

                    @pl.when(l >= _NBUF)
                    def _drain_wb():
                        wb_cp(l, kk).wait()

                    lfull = jnp.full((16,), l, dtype=jnp.int32)

                    def egroup(eg, carry2):
                        e0 = eg * 8
                        for de in range(8):
                            e = e0 + de
                            efull = jnp.full((16,), e, dtype=jnp.int32)
                            splat = plsc.load_gather(pos_v, [lfull, efull])
                            for g in range(BBLK // 16):
                                col = plsc.load_gather(
                                    gbufs[kk], [lane_groups[g], efull])
                                tbufs[kk][e, pl.ds(16 * g, 16)] = col + splat
                        return carry2

                    lax.fori_loop(0, E // 8, egroup, 0)
                    wb_cp(l, kk).start()

            return carry

        lax.fori_loop(0, L, body, 0)
        for k in range(_NBUF):
            wb_cp(L - _NBUF + k, k).wait()

    out = emb_kernel(inputs_t, token_table, pos_table)
    return jnp.transpose(out, (2, 0, 1))


# speedup vs baseline: 1.3777x; 1.3777x over previous
import functools

import jax
import jax.numpy as jnp
from jax import lax
from jax.experimental import pallas as pl
from jax.experimental.pallas import tpu as pltpu
from jax.experimental.pallas import tpu_sc as plsc

_NUM_WORKERS = 32
_NBUF = 3


def kernel(inputs, token_table, pos_table):
    B, L = inputs.shape
    V, E = token_table.shape
    BBLK = B // _NUM_WORKERS

    inputs_t = jnp.swapaxes(inputs, 0, 1)

    mesh = plsc.VectorSubcoreMesh(core_axis_name="c", subcore_axis_name="s")

    @functools.partial(
        pl.kernel,
        mesh=mesh,
        compiler_params=pltpu.CompilerParams(use_tc_tiling_on_sc=False,
                                             needs_layout_passes=False),
        out_type=jax.ShapeDtypeStruct((L, E, B), jnp.float32),
        scratch_types=[
            pltpu.VMEM((L, BBLK), jnp.int32),
            pltpu.VMEM((L, E), jnp.float32),
            [pltpu.VMEM((BBLK, E), jnp.float32)] * _NBUF,
            [pltpu.VMEM((E, BBLK), jnp.float32)] * _NBUF,
            [pltpu.SemaphoreType.DMA] * _NBUF,
            [pltpu.SemaphoreType.DMA] * _NBUF,
        ],
    )
    def emb_kernel(inputs_hbm, table_hbm, pos_hbm, out_hbm,
                   idx_v, pos_v, gbufs, tbufs, gsems, wsems):
        wid = lax.axis_index("s") * 2 + lax.axis_index("c")
        bbase = wid * BBLK

        pltpu.sync_copy(inputs_hbm.at[:, pl.ds(bbase, BBLK)], idx_v)
        pltpu.sync_copy(pos_hbm, pos_v)

        def gather_cp(l, k):
            return pltpu.make_async_copy(
                table_hbm.at[idx_v.at[l, :]], gbufs[k], gsems[k])

        def wb_cp(l, k):
            return pltpu.make_async_copy(
                tbufs[k], out_hbm.at[l, :, pl.ds(bbase, BBLK)], wsems[k])

        for k in range(_NBUF - 1):
            gather_cp(k, k).start()

        lanes = jnp.arange(16, dtype=jnp.int32)
        lane_groups = [lanes + (16 * g) for g in range(BBLK // 16)]

        def body(l, carry):
            k = lax.rem(l, _NBUF)

            @pl.when(l + _NBUF - 1 < L)
            def _fire_ahead():
                kn = lax.rem(l + _NBUF - 1, _NBUF)
                for kk in range(_NBUF):
                    @pl.when(kn == kk)
                    def _fire():
                        gather_cp(l + _NBUF - 1, kk).start()

            for kk in range(_NBUF):
                @pl.when(k == kk)
                def _work():
                    gather_cp(l, kk).wait()

                    @pl.when(l >= _NBUF)
                    def _drain_wb():
                        wb_cp(l, kk).wait()

                    lfull = jnp.full((16,), l, dtype=jnp.int32)

                    @plsc.parallel_loop(0, E, step=1, unroll=8)
                    def _transpose(e):
                        efull = jnp.full((16,), e, dtype=jnp.int32)
                        splat = plsc.load_gather(pos_v, [lfull, efull])
                        for g in range(BBLK // 16):
                            col = plsc.load_gather(
                                gbufs[kk], [lane_groups[g], efull])
                            tbufs[kk][e, pl.ds(16 * g, 16)] = col + splat
                    wb_cp(l, kk).start()

            return carry

        lax.fori_loop(0, L, body, 0)
        for k in range(_NBUF):
            wb_cp(L - _NBUF + k, k).wait()

    out = emb_kernel(inputs_t, token_table, pos_table)
    return jnp.transpose(out, (2, 0, 1))


# speedup vs baseline: 1.8174x; 1.3191x over previous
import functools

import jax
import jax.numpy as jnp
from jax import lax
from jax.experimental import pallas as pl
from jax.experimental.pallas import tpu as pltpu
from jax.experimental.pallas import tpu_sc as plsc

_NUM_WORKERS = 32
_NBUF = 8
_INIT_AHEAD = 4
_GATHER_AHEAD = 2


def kernel(inputs, token_table, pos_table):
    B, L = inputs.shape
    V, E = token_table.shape
    BBLK = B // _NUM_WORKERS

    inputs_t = jnp.swapaxes(inputs, 0, 1)

    mesh = plsc.VectorSubcoreMesh(core_axis_name="c", subcore_axis_name="s")

    @functools.partial(
        pl.kernel,
        mesh=mesh,
        compiler_params=pltpu.CompilerParams(use_tc_tiling_on_sc=False,
                                             needs_layout_passes=False),
        out_type=jax.ShapeDtypeStruct((L, B, E), jnp.float32),
        scratch_types=[
            pltpu.VMEM((L, BBLK), jnp.int32),
            pltpu.VMEM((L, E), jnp.float32),
            pltpu.VMEM_SHARED((L, BBLK // 4, E), jnp.float32),
            [pltpu.VMEM((BBLK, E), jnp.float32)] * _NBUF,
            [pltpu.SemaphoreType.DMA] * _NBUF,
            [pltpu.SemaphoreType.DMA] * _NBUF,
            [pltpu.SemaphoreType.DMA] * _NBUF,
        ],
    )
    def emb_kernel(inputs_hbm, table_hbm, pos_hbm, out_hbm,
                   idx_v, pos_v, posb_sh, bufs, isems, gsems, wsems):
        sid = lax.axis_index("s")
        wid = sid * 2 + lax.axis_index("c")
        bbase = wid * BBLK

        pltpu.sync_copy(inputs_hbm.at[:, pl.ds(bbase, BBLK)], idx_v)
        pltpu.sync_copy(pos_hbm, pos_v)

        def build(l, carry):
            prow = [pos_v[l, pl.ds(16 * q, 16)] for q in range(E // 16)]

            @plsc.parallel_loop(0, BBLK // 4, step=1)
            def _fill(j):
                for q in range(E // 16):
                    bufs[0][j, pl.ds(16 * q, 16)] = prow[q]

            pltpu.sync_copy(bufs[0].at[pl.ds(0, BBLK // 4)], posb_sh.at[l])
            return carry

        lax.fori_loop((sid * L) // 16, ((sid + 1) * L) // 16, build, 0)
        plsc.subcore_barrier()

        def init_cps(l, k):
            q = BBLK // 4
            return [
                pltpu.make_async_copy(
                    posb_sh.at[l], bufs[k].at[pl.ds(i * q, q)], isems[k])
                for i in range(4)
            ]

        def gather_cp(l, k):
            return pltpu.make_async_copy(
                table_hbm.at[idx_v.at[l, :]], bufs[k], gsems[k])

        def wb_cp(l, k):
            return pltpu.make_async_copy(
                bufs[k], out_hbm.at[l, pl.ds(bbase, BBLK)], wsems[k])

        def dispatch(kdyn, fn):
            for kk in range(_NBUF):
                @pl.when(kdyn == kk)
                def _go():
                    fn(kk)
            return None

        for l0 in range(_INIT_AHEAD):
            for cp in init_cps(l0, l0):
                cp.start()
        for l0 in range(_GATHER_AHEAD):
            for cp in init_cps(l0, l0):
                cp.wait()
            gather_cp(l0, l0).start(add=True)

        def body(l, carry):
            @pl.when(l + _INIT_AHEAD < L)
            def _init_ahead():
                li = l + _INIT_AHEAD

                def go(kk):
                    @pl.when(li >= _NBUF)
                    def _drain_wb():
                        wb_cp(li - _NBUF, kk).wait()
                    for cp in init_cps(li, kk):
                        cp.start()

                dispatch(lax.rem(li, _NBUF), go)

            @pl.when(l + _GATHER_AHEAD < L)
            def _gather_ahead():
                lg = l + _GATHER_AHEAD

                def go(kk):
                    for cp in init_cps(lg, kk):
                        cp.wait()
                    gather_cp(lg, kk).start(add=True)

                dispatch(lax.rem(lg, _NBUF), go)

            def go(kk):
                gather_cp(l, kk).wait()
                wb_cp(l, kk).start()

            dispatch(lax.rem(l, _NBUF), go)
            return carry

        lax.fori_loop(0, L, body, 0)
        for l in range(L - _NBUF, L):
            wb_cp(l, l % _NBUF).wait()

    out = emb_kernel(inputs_t, token_table, pos_table)
    return jnp.transpose(out, (1, 0, 2))
